# baseline (speedup 1.0000x reference)
import functools

import jax
import jax.numpy as jnp
from jax import lax
from jax.experimental import pallas as pl
from jax.experimental.pallas import tpu as pltpu
from jax.experimental.pallas import tpu_sc as plsc

N = 10000
E = 320000
D_IN = 128
F1, F2, F3 = 128, 64, 32
BNN = 16

NC, NS = 2, 16
CH = 128
K0, K1 = 80, 80
C0TOT = NS * K0
CHT = NS * (K0 + K1)
EP = CHT * CH
NP = 10112
RPT = NP // NS

_ZCH = [(0, 128), (128, 128), (256, 128), (384, 128), (512, RPT - 512)]


def _make_scatter(F, nbuf, slabs0, slabs1, dtype=jnp.bfloat16, gather=True,
                  resident=False):
    SLM = max(slabs0 + slabs1)
    LW = 32 if dtype == jnp.bfloat16 else 16
    mesh = plsc.VectorSubcoreMesh(core_axis_name="c", subcore_axis_name="s")
    sems = [pltpu.SemaphoreType.DMA] * (2 * nbuf)
    bufs_t = [pltpu.VMEM((CH, F), dtype)] * nbuf
    tab_t = [pltpu.VMEM_SHARED((NP, F), dtype)] if resident else []

    @functools.partial(
        pl.kernel,
        out_type=jax.ShapeDtypeStruct((NC, NP, F), dtype),
        mesh=mesh,
        compiler_params=pltpu.CompilerParams(use_tc_tiling_on_sc=False),
        scratch_types=[
            pltpu.VMEM((SLM, CH), jnp.int32),
            pltpu.VMEM((SLM, CH), jnp.int32),
            pltpu.VMEM_SHARED((NP, F), dtype),
        ] + tab_t + bufs_t + sems,
    )
    def k(h_hbm, src_hbm, dst_hbm, out_hbm, src_v, dst_v, acc_sh, *rest):
        if resident:
            tab_sh, rest = rest[0], rest[1:]
        bufs = rest[:nbuf]
        sg = rest[nbuf:2 * nbuf]
        ss = rest[2 * nbuf:]
        gsrc = tab_sh if resident else h_hbm
        c = lax.axis_index("c")
        s = lax.axis_index("s")
        fill = 1.0 if not gather else 0.0
        zb = bufs[1] if not gather else bufs[0]

        def zbody(i, carry):
            for j in range(F // LW):
                bufs[0][i, pl.ds(j * LW, LW)] = jnp.full((LW,), fill, dtype)
                if zb is not bufs[0]:
                    zb[i, pl.ds(j * LW, LW)] = jnp.zeros((LW,), dtype)
            return carry

        lax.fori_loop(0, CH, zbody, 0)
        base = s * RPT
        for off, sz in _ZCH:
            pltpu.sync_copy(zb.at[pl.ds(0, sz)],
                            acc_sh.at[pl.ds(base + off, sz)])
        if resident:
            NLAST = N - (NS - 1) * RPT

            @pl.when(s < NS - 1)
            def _stage_full():
                pltpu.sync_copy(h_hbm.at[pl.ds(base, RPT)],
                                tab_sh.at[pl.ds(base, RPT)])

            @pl.when(s == NS - 1)
            def _stage_last():
                pltpu.sync_copy(h_hbm.at[pl.ds((NS - 1) * RPT, NLAST)],
                                tab_sh.at[pl.ds((NS - 1) * RPT, NLAST)])
        plsc.subcore_barrier()

        def run_slab(cstart, SL):
            pltpu.sync_copy(src_hbm.at[pl.ds(cstart, SL)],
                            src_v.at[pl.ds(0, SL)])
            pltpu.sync_copy(dst_hbm.at[pl.ds(cstart, SL)],
                            dst_v.at[pl.ds(0, SL)])

            if gather:
                for u in range(nbuf):
                    pltpu.async_copy(gsrc.at[src_v.at[u]], bufs[u], sg[u])

                def body(t, carry):
                    for u in range(nbuf):
                        j = nbuf * t + u
                        pltpu.make_async_copy(
                            gsrc.at[src_v.at[j]], bufs[u], sg[u]).wait()
                        pltpu.async_copy(bufs[u], acc_sh.at[dst_v.at[j]],
                                         ss[u], add=True)

                    @pl.when(t < SL // nbuf - 1)
                    def _prefetch():
                        for u in range(nbuf):
                            jn = nbuf * t + nbuf + u
                            pltpu.make_async_copy(
                                bufs[u], acc_sh.at[dst_v.at[0]], ss[u]).wait()
                            pltpu.async_copy(gsrc.at[src_v.at[jn]],
                                             bufs[u], sg[u])
                    return carry

                lax.fori_loop(0, SL // nbuf, body, 0)
            else:
                def body(t, carry):
                    for u in range(nbuf):
                        j = nbuf * t + u

                        @pl.when(t > 0)
                        def _drain():
                            pltpu.make_async_copy(
                                bufs[0], acc_sh.at[dst_v.at[0]], ss[u]).wait()

                        pltpu.async_copy(bufs[0], acc_sh.at[dst_v.at[j]],
                                         ss[u], add=True)
                    return carry

                lax.fori_loop(0, SL // nbuf, body, 0)

            for u in range(nbuf):
                pltpu.make_async_copy(bufs[0], acc_sh.at[dst_v.at[0]],
                                      ss[u]).wait()

        @pl.when(c == 0)
        def _core0():
            off = 0
            for SL in slabs0:
                run_slab(s * K0 + off, SL)
                off += SL

        @pl.when(c == 1)
        def _core1():
            off = 0
            for SL in slabs1:
                run_slab(C0TOT + s * K1 + off, SL)
                off += SL

        plsc.subcore_barrier()
        pltpu.sync_copy(acc_sh.at[pl.ds(base, RPT)],
                        out_hbm.at[c, pl.ds(base, RPT)])

    return k


_scatter16 = _make_scatter(16, 4, [K0], [K1], dtype=jnp.float32, gather=False)
_scatter128 = _make_scatter(F1, 2, [K0], [K1], resident=True)
_scatter64 = _make_scatter(F2, 4, [K0], [K1], resident=True)
_scatter32 = _make_scatter(F3, 4, [K0], [K1], resident=True)

_R = 2000


def _prologue(parts16, x, W1):
    def body(p_ref, x_ref, w_ref, dinv_ref, h1p_ref):
        deg = p_ref[0] + p_ref[1] + 1.0
        dinv = lax.rsqrt(deg)
        dinv_ref[...] = dinv
        h = jnp.dot(x_ref[...], w_ref[...], preferred_element_type=jnp.float32)
        h1p_ref[...] = (h * dinv[:, :1]).astype(jnp.bfloat16)

    return pl.pallas_call(
        body,
        grid=(N // _R,),
        in_specs=[
            pl.BlockSpec((2, _R, 16), lambda i: (0, i, 0)),
            pl.BlockSpec((_R, D_IN), lambda i: (i, 0)),
            pl.BlockSpec((D_IN, F1), lambda i: (0, 0)),
        ],
        out_specs=[
            pl.BlockSpec((_R, 16), lambda i: (i, 0)),
            pl.BlockSpec((_R, F1), lambda i: (i, 0)),
        ],
        out_shape=[
            jax.ShapeDtypeStruct((N, 16), jnp.float32),
            jax.ShapeDtypeStruct((N, F1), jnp.bfloat16),
        ],
    )(parts16, x, W1)


def _epilogue(parts, hp, dinv16, b, Wn):
    F = hp.shape[1]
    Fn = Wn.shape[1]

    def body(p_ref, hp_ref, dinv_ref, b_ref, w_ref, o_ref):
        d = dinv_ref[...][:, :1]
        ssum = (p_ref[0].astype(jnp.float32) + p_ref[1].astype(jnp.float32)
                + hp_ref[...].astype(jnp.float32))
        z = d * ssum + b_ref[...]
        h = jnp.maximum(z, 0.0)
        o = d * jnp.dot(h, w_ref[...], preferred_element_type=jnp.float32)
        o_ref[...] = o.astype(jnp.bfloat16)

    return pl.pallas_call(
        body,
        grid=(N // _R,),
        in_specs=[
            pl.BlockSpec((2, _R, F), lambda i: (0, i, 0)),
            pl.BlockSpec((_R, F), lambda i: (i, 0)),
            pl.BlockSpec((_R, 16), lambda i: (i, 0)),
            pl.BlockSpec((1, F), lambda i: (0, 0)),
            pl.BlockSpec((F, Fn), lambda i: (0, 0)),
        ],
        out_specs=pl.BlockSpec((_R, Fn), lambda i: (i, 0)),
        out_shape=jax.ShapeDtypeStruct((N, Fn), jnp.bfloat16),
    )(parts, hp, dinv16, b, Wn)


def _final_a(parts, h3p, dinv16, b3):
    def body(p_ref, hp_ref, dinv_ref, b_ref, h3_ref, cs_ref):
        i = pl.program_id(0)
        d = dinv_ref[...][:, :1]
        h3 = d * (p_ref[0].astype(jnp.float32) + p_ref[1].astype(jnp.float32)
                  + hp_ref[...].astype(jnp.float32)) + b_ref[...]
        h3_ref[...] = h3
        csum = jnp.sum(h3, axis=0, keepdims=True)

        @pl.when(i == 0)
        def _init():
            cs_ref[...] = csum

        @pl.when(i > 0)
        def _acc():
            cs_ref[...] += csum

    return pl.pallas_call(
        body,
        grid=(N // _R,),
        in_specs=[
            pl.BlockSpec((2, _R, F3), lambda i: (0, i, 0)),
            pl.BlockSpec((_R, F3), lambda i: (i, 0)),
            pl.BlockSpec((_R, 16), lambda i: (i, 0)),
            pl.BlockSpec((1, F3), lambda i: (0, 0)),
        ],
        out_specs=[
            pl.BlockSpec((_R, F3), lambda i: (i, 0)),
            pl.BlockSpec((1, F3), lambda i: (0, 0)),
        ],
        out_shape=[
            jax.ShapeDtypeStruct((N, F3), jnp.float32),
            jax.ShapeDtypeStruct((1, F3), jnp.float32),
        ],
    )(parts, h3p, dinv16, b3)


def _final_b(h3, cs, Watt, fcW, fcb, sW, sb):
    def body(h3_ref, cs_ref, watt_ref, fcw_ref, fcb_ref, sw_ref, sb_ref,
             o_ref, rep_ref):
        i = pl.program_id(0)
        tg = jnp.tanh(jnp.dot(cs_ref[...] * (1.0 / N), watt_ref[...],
                              preferred_element_type=jnp.float32))
        h3 = h3_ref[...]
        sig = 1.0 / (1.0 + jnp.exp(-jnp.sum(h3 * tg, axis=1, keepdims=True)))
        rblk = jnp.sum(h3 * sig, axis=0, keepdims=True)

        @pl.when(i == 0)
        def _init():
            rep_ref[...] = rblk

        @pl.when(i > 0)
        def _acc():
            rep_ref[...] += rblk

        @pl.when(i == N // _R - 1)
        def _head():
            s2 = jnp.maximum(
                jnp.dot(rep_ref[...], fcw_ref[...],
                        preferred_element_type=jnp.float32) + fcb_ref[...], 0.0)
            o_ref[...] = 1.0 / (1.0 + jnp.exp(
                -(jnp.dot(s2, sw_ref[...], preferred_element_type=jnp.float32)
                  + sb_ref[...])))

    return pl.pallas_call(
        body,
        grid=(N // _R,),
        in_specs=[
            pl.BlockSpec((_R, F3), lambda i: (i, 0)),
            pl.BlockSpec((1, F3), lambda i: (0, 0)),
            pl.BlockSpec((F3, F3), lambda i: (0, 0)),
            pl.BlockSpec((F3, BNN), lambda i: (0, 0)),
            pl.BlockSpec((1, BNN), lambda i: (0, 0)),
            pl.BlockSpec((BNN, 1), lambda i: (0, 0)),
            pl.BlockSpec((1, 1), lambda i: (0, 0)),
        ],
        out_specs=pl.BlockSpec((1, 1), lambda i: (0, 0)),
        out_shape=jax.ShapeDtypeStruct((1, 1), jnp.float32),
        scratch_shapes=[pltpu.VMEM((1, F3), jnp.float32)],
    )(h3, cs, Watt, fcW, fcb, sW, sb)


def kernel(features_1, edge_index_1, W1, b1, W2, b2, W3, b3, Watt, fcW, fcb, sW, sb):
    ei = edge_index_1.astype(jnp.int32)
    src, dst = ei[0], ei[1]
    pad = EP - E
    srcp = jnp.concatenate([src, jnp.zeros((pad,), jnp.int32)]).reshape(CHT, CH)
    dstp = jnp.concatenate([dst, jnp.full((pad,), N, jnp.int32)]).reshape(CHT, CH)

    dummy16 = jnp.zeros((N, 16), jnp.float32)
    degp = _scatter16(dummy16, srcp, dstp)[:, :N]

    dinv16, h1p = _prologue(degp, features_1, W1)
    s1 = _scatter128(h1p, srcp, dstp)[:, :N]
    h2p = _epilogue(s1, h1p, dinv16, b1.reshape(1, F1), W2)
    s2 = _scatter64(h2p, srcp, dstp)[:, :N]
    h3p = _epilogue(s2, h2p, dinv16, b2.reshape(1, F2), W3)
    s3 = _scatter32(h3p, srcp, dstp)[:, :N]
    h3, cs = _final_a(s3, h3p, dinv16, b3.reshape(1, F3))
    return _final_b(h3, cs, Watt, fcW, fcb.reshape(1, BNN), sW,
                    sb.reshape(1, 1))


# speedup vs baseline: 1.0029x; 1.0029x over previous
import functools

import jax
import jax.numpy as jnp
from jax import lax
from jax.experimental import pallas as pl
from jax.experimental.pallas import tpu as pltpu
from jax.experimental.pallas import tpu_sc as plsc

N = 10000
E = 320000
D_IN = 128
F1, F2, F3 = 128, 64, 32
BNN = 16

NC, NS = 2, 16
CH = 128
K0, K1 = 81, 78
C0TOT = NS * K0
CHT = NS * (K0 + K1)
EP = CHT * CH
NP = 10112
RPT = NP // NS

_ZCH = [(0, 128), (128, 128), (256, 128), (384, 128), (512, RPT - 512)]


def _make_scatter(F, nbuf, slabs0, slabs1, dtype=jnp.bfloat16, gather=True,
                  resident=False):
    SLM = max(slabs0 + slabs1)
    LW = 32 if dtype == jnp.bfloat16 else 16
    mesh = plsc.VectorSubcoreMesh(core_axis_name="c", subcore_axis_name="s")
    sems = [pltpu.SemaphoreType.DMA] * (2 * nbuf)
    bufs_t = [pltpu.VMEM((CH, F), dtype)] * nbuf
    tab_t = [pltpu.VMEM_SHARED((NP, F), dtype)] if resident else []

    @functools.partial(
        pl.kernel,
        out_type=jax.ShapeDtypeStruct((NC, NP, F), dtype),
        mesh=mesh,
        compiler_params=pltpu.CompilerParams(use_tc_tiling_on_sc=False),
        scratch_types=[
            pltpu.VMEM((SLM, CH), jnp.int32),
            pltpu.VMEM((SLM, CH), jnp.int32),
            pltpu.VMEM_SHARED((NP, F), dtype),
        ] + tab_t + bufs_t + sems,
    )
    def k(h_hbm, src_hbm, dst_hbm, out_hbm, src_v, dst_v, acc_sh, *rest):
        if resident:
            tab_sh, rest = rest[0], rest[1:]
        bufs = rest[:nbuf]
        sg = rest[nbuf:2 * nbuf]
        ss = rest[2 * nbuf:]
        gsrc = tab_sh if resident else h_hbm
        c = lax.axis_index("c")
        s = lax.axis_index("s")
        fill = 1.0 if not gather else 0.0
        zb = bufs[1] if not gather else bufs[0]

        def zbody(i, carry):
            for j in range(F // LW):
                bufs[0][i, pl.ds(j * LW, LW)] = jnp.full((LW,), fill, dtype)
                if zb is not bufs[0]:
                    zb[i, pl.ds(j * LW, LW)] = jnp.zeros((LW,), dtype)
            return carry

        lax.fori_loop(0, CH, zbody, 0)
        base = s * RPT
        for off, sz in _ZCH:
            pltpu.sync_copy(zb.at[pl.ds(0, sz)],
                            acc_sh.at[pl.ds(base + off, sz)])
        if resident:
            NLAST = N - (NS - 1) * RPT

            @pl.when(s < NS - 1)
            def _stage_full():
                pltpu.sync_copy(h_hbm.at[pl.ds(base, RPT)],
                                tab_sh.at[pl.ds(base, RPT)])

            @pl.when(s == NS - 1)
            def _stage_last():
                pltpu.sync_copy(h_hbm.at[pl.ds((NS - 1) * RPT, NLAST)],
                                tab_sh.at[pl.ds((NS - 1) * RPT, NLAST)])
        plsc.subcore_barrier()

        def run_slab(cstart, SL):
            pltpu.sync_copy(src_hbm.at[pl.ds(cstart, SL)],
                            src_v.at[pl.ds(0, SL)])
            pltpu.sync_copy(dst_hbm.at[pl.ds(cstart, SL)],
                            dst_v.at[pl.ds(0, SL)])

            if gather:
                for u in range(nbuf):
                    pltpu.async_copy(gsrc.at[src_v.at[u]], bufs[u], sg[u])

                def body(t, carry):
                    for u in range(nbuf):
                        j = nbuf * t + u
                        pltpu.make_async_copy(
                            gsrc.at[src_v.at[j]], bufs[u], sg[u]).wait()
                        pltpu.async_copy(bufs[u], acc_sh.at[dst_v.at[j]],
                                         ss[u], add=True)

                    @pl.when(t < SL // nbuf - 1)
                    def _prefetch():
                        for u in range(nbuf):
                            jn = nbuf * t + nbuf + u
                            pltpu.make_async_copy(
                                bufs[u], acc_sh.at[dst_v.at[0]], ss[u]).wait()
                            pltpu.async_copy(gsrc.at[src_v.at[jn]],
                                             bufs[u], sg[u])
                    return carry

                lax.fori_loop(0, SL // nbuf, body, 0)
            else:
                def body(t, carry):
                    for u in range(nbuf):
                        j = nbuf * t + u

                        @pl.when(t > 0)
                        def _drain():
                            pltpu.make_async_copy(
                                bufs[0], acc_sh.at[dst_v.at[0]], ss[u]).wait()

                        pltpu.async_copy(bufs[0], acc_sh.at[dst_v.at[j]],
                                         ss[u], add=True)
                    return carry

                lax.fori_loop(0, SL // nbuf, body, 0)

            for u in range(nbuf):
                pltpu.make_async_copy(bufs[0], acc_sh.at[dst_v.at[0]],
                                      ss[u]).wait()

        @pl.when(c == 0)
        def _core0():
            off = 0
            for SL in slabs0:
                run_slab(s * K0 + off, SL)
                off += SL

        @pl.when(c == 1)
        def _core1():
            off = 0
            for SL in slabs1:
                run_slab(C0TOT + s * K1 + off, SL)
                off += SL

        plsc.subcore_barrier()
        pltpu.sync_copy(acc_sh.at[pl.ds(base, RPT)],
                        out_hbm.at[c, pl.ds(base, RPT)])

    return k


_scatter16 = _make_scatter(16, 3, [K0], [K1], dtype=jnp.float32, gather=False)
_scatter128 = _make_scatter(F1, 3, [K0], [K1], resident=True)
_scatter64 = _make_scatter(F2, 3, [K0], [K1], resident=True)
_scatter32 = _make_scatter(F3, 3, [K0], [K1], resident=True)

_R = 2000


def _prologue(parts16, x, W1):
    def body(p_ref, x_ref, w_ref, dinv_ref, h1p_ref):
        deg = p_ref[0] + p_ref[1] + 1.0
        dinv = lax.rsqrt(deg)
        dinv_ref[...] = dinv
        h = jnp.dot(x_ref[...], w_ref[...], preferred_element_type=jnp.float32)
        h1p_ref[...] = (h * dinv[:, :1]).astype(jnp.bfloat16)

    return pl.pallas_call(
        body,
        grid=(N // _R,),
        in_specs=[
            pl.BlockSpec((2, _R, 16), lambda i: (0, i, 0)),
            pl.BlockSpec((_R, D_IN), lambda i: (i, 0)),
            pl.BlockSpec((D_IN, F1), lambda i: (0, 0)),
        ],
        out_specs=[
            pl.BlockSpec((_R, 16), lambda i: (i, 0)),
            pl.BlockSpec((_R, F1), lambda i: (i, 0)),
        ],
        out_shape=[
            jax.ShapeDtypeStruct((N, 16), jnp.float32),
            jax.ShapeDtypeStruct((N, F1), jnp.bfloat16),
        ],
    )(parts16, x, W1)


def _epilogue(parts, hp, dinv16, b, Wn):
    F = hp.shape[1]
    Fn = Wn.shape[1]

    def body(p_ref, hp_ref, dinv_ref, b_ref, w_ref, o_ref):
        d = dinv_ref[...][:, :1]
        ssum = (p_ref[0].astype(jnp.float32) + p_ref[1].astype(jnp.float32)
                + hp_ref[...].astype(jnp.float32))
        z = d * ssum + b_ref[...]
        h = jnp.maximum(z, 0.0)
        o = d * jnp.dot(h, w_ref[...], preferred_element_type=jnp.float32)
        o_ref[...] = o.astype(jnp.bfloat16)

    return pl.pallas_call(
        body,
        grid=(N // _R,),
        in_specs=[
            pl.BlockSpec((2, _R, F), lambda i: (0, i, 0)),
            pl.BlockSpec((_R, F), lambda i: (i, 0)),
            pl.BlockSpec((_R, 16), lambda i: (i, 0)),
            pl.BlockSpec((1, F), lambda i: (0, 0)),
            pl.BlockSpec((F, Fn), lambda i: (0, 0)),
        ],
        out_specs=pl.BlockSpec((_R, Fn), lambda i: (i, 0)),
        out_shape=jax.ShapeDtypeStruct((N, Fn), jnp.bfloat16),
    )(parts, hp, dinv16, b, Wn)


def _final(parts, h3p, dinv16, b3, Watt, fcW, fcb, sW, sb):
    def body(p_ref, hp_ref, dinv_ref, b_ref, watt_ref, fcw_ref, fcb_ref,
             sw_ref, sb_ref, o_ref):
        d = dinv_ref[...][:, :1]
        h3 = d * (p_ref[0].astype(jnp.float32) + p_ref[1].astype(jnp.float32)
                  + hp_ref[...].astype(jnp.float32)) + b_ref[...]
        gc = jnp.dot(jnp.mean(h3, axis=0, keepdims=True), watt_ref[...])
        tg = jnp.tanh(gc)
        sig = 1.0 / (1.0 + jnp.exp(-jnp.sum(h3 * tg, axis=1, keepdims=True)))
        rep = jnp.sum(h3 * sig, axis=0, keepdims=True)
        s2 = jnp.maximum(
            jnp.dot(rep, fcw_ref[...], preferred_element_type=jnp.float32)
            + fcb_ref[...], 0.0)
        o = 1.0 / (1.0 + jnp.exp(
            -(jnp.dot(s2, sw_ref[...], preferred_element_type=jnp.float32)
              + sb_ref[...])))
        o_ref[...] = o

    return pl.pallas_call(
        body,
        out_shape=jax.ShapeDtypeStruct((1, 1), jnp.float32),
    )(parts, h3p, dinv16, b3, Watt, fcW, fcb, sW, sb)


def kernel(features_1, edge_index_1, W1, b1, W2, b2, W3, b3, Watt, fcW, fcb, sW, sb):
    ei = edge_index_1.astype(jnp.int32)
    src, dst = ei[0], ei[1]
    pad = EP - E
    srcp = jnp.concatenate([src, jnp.zeros((pad,), jnp.int32)]).reshape(CHT, CH)
    dstp = jnp.concatenate([dst, jnp.full((pad,), N, jnp.int32)]).reshape(CHT, CH)

    dummy16 = jnp.zeros((N, 16), jnp.float32)
    degp = _scatter16(dummy16, srcp, dstp)[:, :N]

    dinv16, h1p = _prologue(degp, features_1, W1)
    s1 = _scatter128(h1p, srcp, dstp)[:, :N]
    h2p = _epilogue(s1, h1p, dinv16, b1.reshape(1, F1), W2)
    s2 = _scatter64(h2p, srcp, dstp)[:, :N]
    h3p = _epilogue(s2, h2p, dinv16, b2.reshape(1, F2), W3)
    s3 = _scatter32(h3p, srcp, dstp)[:, :N]
    return _final(s3, h3p, dinv16, b3.reshape(1, F3), Watt, fcW,
                  fcb.reshape(1, BNN), sW, sb.reshape(1, 1))


# speedup vs baseline: 1.0170x; 1.0140x over previous
import functools

import jax
import jax.numpy as jnp
from jax import lax
from jax.experimental import pallas as pl
from jax.experimental.pallas import tpu as pltpu
from jax.experimental.pallas import tpu_sc as plsc

N = 10000
E = 320000
D_IN = 128
F1, F2, F3 = 128, 64, 32
BNN = 16

NC, NS = 2, 16
CH = 128
K0, K1 = 80, 80
C0TOT = NS * K0
CHT = NS * (K0 + K1)
EP = CHT * CH
NP = 10112
RPT = NP // NS

_ZCH = [(0, 128), (128, 128), (256, 128), (384, 128), (512, RPT - 512)]


def _make_scatter(F, nbuf, slabs0, slabs1, dtype=jnp.bfloat16, gather=True,
                  resident=False):
    SLM = max(slabs0 + slabs1)
    LW = 32 if dtype == jnp.bfloat16 else 16
    mesh = plsc.VectorSubcoreMesh(core_axis_name="c", subcore_axis_name="s")
    sems = [pltpu.SemaphoreType.DMA] * (2 * nbuf)
    bufs_t = [pltpu.VMEM((CH, F), dtype)] * nbuf
    tab_t = [pltpu.VMEM_SHARED((NP, F), dtype)] if resident else []

    @functools.partial(
        pl.kernel,
        out_type=jax.ShapeDtypeStruct((NC, NP, F), dtype),
        mesh=mesh,
        compiler_params=pltpu.CompilerParams(use_tc_tiling_on_sc=False),
        scratch_types=[
            pltpu.VMEM((SLM, CH), jnp.int32),
            pltpu.VMEM((SLM, CH), jnp.int32),
            pltpu.VMEM_SHARED((NP, F), dtype),
        ] + tab_t + bufs_t + sems,
    )
    def k(h_hbm, src_hbm, dst_hbm, out_hbm, src_v, dst_v, acc_sh, *rest):
        if resident:
            tab_sh, rest = rest[0], rest[1:]
        bufs = rest[:nbuf]
        sg = rest[nbuf:2 * nbuf]
        ss = rest[2 * nbuf:]
        gsrc = tab_sh if resident else h_hbm
        c = lax.axis_index("c")
        s = lax.axis_index("s")
        fill = 1.0 if not gather else 0.0
        zb = bufs[1] if not gather else bufs[0]

        def zbody(i, carry):
            for j in range(F // LW):
                bufs[0][i, pl.ds(j * LW, LW)] = jnp.full((LW,), fill, dtype)
                if zb is not bufs[0]:
                    zb[i, pl.ds(j * LW, LW)] = jnp.zeros((LW,), dtype)
            return carry

        lax.fori_loop(0, CH, zbody, 0)
        base = s * RPT
        for off, sz in _ZCH:
            pltpu.sync_copy(zb.at[pl.ds(0, sz)],
                            acc_sh.at[pl.ds(base + off, sz)])
        if resident:
            NLAST = N - (NS - 1) * RPT

            @pl.when(s < NS - 1)
            def _stage_full():
                pltpu.sync_copy(h_hbm.at[pl.ds(base, RPT)],
                                tab_sh.at[pl.ds(base, RPT)])

            @pl.when(s == NS - 1)
            def _stage_last():
                pltpu.sync_copy(h_hbm.at[pl.ds((NS - 1) * RPT, NLAST)],
                                tab_sh.at[pl.ds((NS - 1) * RPT, NLAST)])
        plsc.subcore_barrier()

        def run_slab(cstart, SL):
            pltpu.sync_copy(src_hbm.at[pl.ds(cstart, SL)],
                            src_v.at[pl.ds(0, SL)])
            pltpu.sync_copy(dst_hbm.at[pl.ds(cstart, SL)],
                            dst_v.at[pl.ds(0, SL)])

            if gather:
                for u in range(nbuf):
                    pltpu.async_copy(gsrc.at[src_v.at[u]], bufs[u], sg[u])

                def body(t, carry):
                    for u in range(nbuf):
                        j = nbuf * t + u
                        pltpu.make_async_copy(
                            gsrc.at[src_v.at[j]], bufs[u], sg[u]).wait()
                        pltpu.async_copy(bufs[u], acc_sh.at[dst_v.at[j]],
                                         ss[u], add=True)

                    @pl.when(t < SL // nbuf - 1)
                    def _prefetch():
                        for u in range(nbuf):
                            jn = nbuf * t + nbuf + u
                            pltpu.make_async_copy(
                                bufs[u], acc_sh.at[dst_v.at[0]], ss[u]).wait()
                            pltpu.async_copy(gsrc.at[src_v.at[jn]],
                                             bufs[u], sg[u])
                    return carry

                lax.fori_loop(0, SL // nbuf, body, 0)
            else:
                def body(t, carry):
                    for u in range(nbuf):
                        j = nbuf * t + u

                        @pl.when(t > 0)
                        def _drain():
                            pltpu.make_async_copy(
                                bufs[0], acc_sh.at[dst_v.at[0]], ss[u]).wait()

                        pltpu.async_copy(bufs[0], acc_sh.at[dst_v.at[j]],
                                         ss[u], add=True)
                    return carry

                lax.fori_loop(0, SL // nbuf, body, 0)

            for u in range(nbuf):
                pltpu.make_async_copy(bufs[0], acc_sh.at[dst_v.at[0]],
                                      ss[u]).wait()

        @pl.when(c == 0)
        def _core0():
            off = 0
            for SL in slabs0:
                run_slab(s * K0 + off, SL)
                off += SL

        @pl.when(c == 1)
        def _core1():
            off = 0
            for SL in slabs1:
                run_slab(C0TOT + s * K1 + off, SL)
                off += SL

        plsc.subcore_barrier()
        pltpu.sync_copy(acc_sh.at[pl.ds(base, RPT)],
                        out_hbm.at[c, pl.ds(base, RPT)])

    return k


_scatter16 = _make_scatter(16, 4, [K0], [K1], dtype=jnp.float32, gather=False)
_scatter128 = _make_scatter(F1, 2, [K0], [K1], resident=True)
_scatter64 = _make_scatter(F2, 4, [K0], [K1], resident=True)
_scatter32 = _make_scatter(F3, 4, [K0], [K1], resident=True)

_R = 2000


def _prologue(parts16, x, W1):
    def body(p_ref, x_ref, w_ref, dinv_ref, h1p_ref):
        deg = p_ref[0] + p_ref[1] + 1.0
        dinv = lax.rsqrt(deg)
        dinv_ref[...] = dinv
        h = jnp.dot(x_ref[...], w_ref[...], preferred_element_type=jnp.float32)
        h1p_ref[...] = (h * dinv[:, :1]).astype(jnp.bfloat16)

    return pl.pallas_call(
        body,
        grid=(N // _R,),
        in_specs=[
            pl.BlockSpec((2, _R, 16), lambda i: (0, i, 0)),
            pl.BlockSpec((_R, D_IN), lambda i: (i, 0)),
            pl.BlockSpec((D_IN, F1), lambda i: (0, 0)),
        ],
        out_specs=[
            pl.BlockSpec((_R, 16), lambda i: (i, 0)),
            pl.BlockSpec((_R, F1), lambda i: (i, 0)),
        ],
        out_shape=[
            jax.ShapeDtypeStruct((N, 16), jnp.float32),
            jax.ShapeDtypeStruct((N, F1), jnp.bfloat16),
        ],
    )(parts16, x, W1)


def _epilogue(parts, hp, dinv16, b, Wn):
    F = hp.shape[1]
    Fn = Wn.shape[1]

    def body(p_ref, hp_ref, dinv_ref, b_ref, w_ref, o_ref):
        d = dinv_ref[...][:, :1]
        ssum = (p_ref[0].astype(jnp.float32) + p_ref[1].astype(jnp.float32)
                + hp_ref[...].astype(jnp.float32))
        z = d * ssum + b_ref[...]
        h = jnp.maximum(z, 0.0)
        o = d * jnp.dot(h, w_ref[...], preferred_element_type=jnp.float32)
        o_ref[...] = o.astype(jnp.bfloat16)

    return pl.pallas_call(
        body,
        grid=(N // _R,),
        in_specs=[
            pl.BlockSpec((2, _R, F), lambda i: (0, i, 0)),
            pl.BlockSpec((_R, F), lambda i: (i, 0)),
            pl.BlockSpec((_R, 16), lambda i: (i, 0)),
            pl.BlockSpec((1, F), lambda i: (0, 0)),
            pl.BlockSpec((F, Fn), lambda i: (0, 0)),
        ],
        out_specs=pl.BlockSpec((_R, Fn), lambda i: (i, 0)),
        out_shape=jax.ShapeDtypeStruct((N, Fn), jnp.bfloat16),
    )(parts, hp, dinv16, b, Wn)


def _final(parts, h3p, dinv16, b3, Watt, fcW, fcb, sW, sb):
    def body(p_ref, hp_ref, dinv_ref, b_ref, watt_ref, fcw_ref, fcb_ref,
             sw_ref, sb_ref, o_ref):
        d = dinv_ref[...][:, :1]
        h3 = d * (p_ref[0].astype(jnp.float32) + p_ref[1].astype(jnp.float32)
                  + hp_ref[...].astype(jnp.float32)) + b_ref[...]
        gc = jnp.dot(jnp.mean(h3, axis=0, keepdims=True), watt_ref[...])
        tg = jnp.tanh(gc)
        sig = 1.0 / (1.0 + jnp.exp(-jnp.sum(h3 * tg, axis=1, keepdims=True)))
        rep = jnp.sum(h3 * sig, axis=0, keepdims=True)
        s2 = jnp.maximum(
            jnp.dot(rep, fcw_ref[...], preferred_element_type=jnp.float32)
            + fcb_ref[...], 0.0)
        o = 1.0 / (1.0 + jnp.exp(
            -(jnp.dot(s2, sw_ref[...], preferred_element_type=jnp.float32)
              + sb_ref[...])))
        o_ref[...] = o

    return pl.pallas_call(
        body,
        out_shape=jax.ShapeDtypeStruct((1, 1), jnp.float32),
    )(parts, h3p, dinv16, b3, Watt, fcW, fcb, sW, sb)


def kernel(features_1, edge_index_1, W1, b1, W2, b2, W3, b3, Watt, fcW, fcb, sW, sb):
    ei = edge_index_1.astype(jnp.int32)
    src, dst = ei[0], ei[1]
    pad = EP - E
    srcp = jnp.concatenate([src, jnp.zeros((pad,), jnp.int32)]).reshape(CHT, CH)
    dstp = jnp.concatenate([dst, jnp.full((pad,), N, jnp.int32)]).reshape(CHT, CH)

    dummy16 = jnp.zeros((N, 16), jnp.float32)
    degp = _scatter16(dummy16, srcp, dstp)[:, :N]

    dinv16, h1p = _prologue(degp, features_1, W1)
    s1 = _scatter128(h1p, srcp, dstp)[:, :N]
    h2p = _epilogue(s1, h1p, dinv16, b1.reshape(1, F1), W2)
    s2 = _scatter64(h2p, srcp, dstp)[:, :N]
    h3p = _epilogue(s2, h2p, dinv16, b2.reshape(1, F2), W3)
    s3 = _scatter32(h3p, srcp, dstp)[:, :N]
    return _final(s3, h3p, dinv16, b3.reshape(1, F3), Watt, fcW,
                  fcb.reshape(1, BNN), sW, sb.reshape(1, 1))
